# baseline (speedup 1.0000x reference)
import functools

import jax
import jax.numpy as jnp
import numpy as np
from jax import lax
from jax.experimental import pallas as pl
from jax.experimental.pallas import tpu as pltpu
from jax.experimental.pallas import tpu_sc as plsc

NUM_EMB = [100000] * 26
EMBED_DIM = 32
BATCH = 4096
N_FIELDS = 26
N_CONCAT = N_FIELDS + 1
BB = 512


def _dense_body(dense_ref, sparse_ref, wb0_ref, bb0_ref, wb1_ref, bb1_ref,
                wb2_ref, bb2_ref, ahat_ref, wt0d_ref, bt0_ref, wt1_ref,
                bt1_ref, wt2_ref, bt2_ref, out_ref, xs_ref):
    xd = dense_ref[...]
    h = jnp.maximum(xd @ wb0_ref[...] + bb0_ref[...], 0.0)
    h = jnp.maximum(h @ wb1_ref[...] + bb1_ref[...], 0.0)
    dout = h @ wb2_ref[...] + bb2_ref[...]

    xs_ref[:, 0, :] = dout
    xs_ref[:, 1:, :] = sparse_ref[...]
    x = xs_ref[...]

    g = lax.dot_general(x, x, (((2,), (2,)), ((0,), (0,))),
                        preferred_element_type=jnp.float32)

    t = dout @ wt0d_ref[...] + bt0_ref[...]
    for i in range(N_CONCAT):
        t += g[:, i, :] @ ahat_ref[i]
    h1 = jnp.maximum(t, 0.0)
    h2 = jnp.maximum(h1 @ wt1_ref[...] + bt1_ref[...], 0.0)
    out_ref[...] = h2 @ wt2_ref[...] + bt2_ref[...]


@jax.jit
def _dense_tail(dense_x, sparse_out, Wb0p, bb0, Wb1, bb1, Wb2, bb2,
                Ahat, Wt0d, bt0, Wt1, bt1, Wt2, bt2):
    nblk = BATCH // BB
    whole = lambda *shape: pl.BlockSpec(shape, lambda i: (0,) * len(shape))
    out = pl.pallas_call(
        _dense_body,
        grid=(nblk,),
        in_specs=[
            pl.BlockSpec((BB, 128), lambda i: (i, 0)),
            pl.BlockSpec((BB, N_FIELDS, EMBED_DIM), lambda i: (i, 0, 0)),
            whole(128, 512), whole(1, 512),
            whole(512, 256), whole(1, 256),
            whole(256, 32), whole(1, 32),
            whole(N_CONCAT, N_CONCAT, 512),
            whole(32, 512), whole(1, 512),
            whole(512, 256), whole(1, 256),
            whole(256, 1), whole(1, 1),
        ],
        out_specs=pl.BlockSpec((BB, 1), lambda i: (i, 0)),
        out_shape=jax.ShapeDtypeStruct((BATCH, 1), jnp.float32),
        scratch_shapes=[pltpu.VMEM((BB, N_CONCAT, EMBED_DIM), jnp.float32)],
    )(dense_x, sparse_out, Wb0p, bb0, Wb1, bb1, Wb2, bb2,
      Ahat, Wt0d, bt0, Wt1, bt1, Wt2, bt2)
    return out[:, 0]


N_LOOKUP = BATCH * N_FIELDS
NW = 32
BPW = N_LOOKUP // NW
_SC_MESH = plsc.VectorSubcoreMesh(core_axis_name="c", subcore_axis_name="s")


@functools.partial(
    pl.kernel,
    mesh=_SC_MESH,
    out_type=jax.ShapeDtypeStruct((N_LOOKUP, EMBED_DIM), jnp.float32),
    scratch_types=[
        pltpu.VMEM((BPW,), jnp.int32),
        pltpu.VMEM((BPW, EMBED_DIM), jnp.float32),
        pltpu.SemaphoreType.DMA,
    ],
    compiler_params=pltpu.CompilerParams(use_tc_tiling_on_sc=False),
)
def _sc_gather(table_hbm, idx_hbm, out_hbm, idx_v, rows_v, sem):
    wid = lax.axis_index("s") * 2 + lax.axis_index("c")
    base = wid * BPW
    pltpu.sync_copy(idx_hbm.at[pl.ds(base, BPW)], idx_v)
    pltpu.async_copy(table_hbm.at[idx_v], rows_v, sem).wait()
    pltpu.sync_copy(rows_v, out_hbm.at[pl.ds(base, BPW)])


def kernel(dense_x, sparse_x, embedding_table,
           Wb0, bb0, Wb1, bb1, Wb2, bb2,
           Wt0, bt0, Wt1, bt1, Wt2, bt2):
    offsets = jnp.asarray(np.concatenate([[0], np.cumsum(NUM_EMB)[:-1]]),
                          dtype=sparse_x.dtype)
    indices = (sparse_x + offsets[None, :]).reshape(-1).astype(jnp.int32)
    sparse_out = _sc_gather(embedding_table, indices).reshape(
        BATCH, N_FIELDS, EMBED_DIM)

    dense_xp = jnp.pad(dense_x, ((0, 0), (0, 128 - dense_x.shape[1])))
    Wb0p = jnp.pad(Wb0, ((0, 128 - Wb0.shape[0]), (0, 0)))

    iu = np.triu_indices(N_CONCAT, k=1)
    Ahat = jnp.zeros((N_CONCAT, N_CONCAT, 512), jnp.float32)
    Ahat = Ahat.at[iu[0], iu[1]].set(Wt0[:351])
    Ahat = 0.5 * (Ahat + jnp.transpose(Ahat, (1, 0, 2)))
    Wt0d = Wt0[351:]

    out = _dense_tail(dense_xp, sparse_out, Wb0p, bb0[None, :], Wb1,
                      bb1[None, :], Wb2, bb2[None, :], Ahat, Wt0d,
                      bt0[None, :], Wt1, bt1[None, :], Wt2, bt2[None, :])
    return out


# speedup vs baseline: 3.6378x; 3.6378x over previous
import jax
import jax.numpy as jnp
import numpy as np
from jax import lax
from jax.experimental import pallas as pl
from jax.experimental.pallas import tpu as pltpu

NUM_EMB = [100000] * 26
EMBED_DIM = 32
BATCH = 4096
N_FIELDS = 26
N_CONCAT = N_FIELDS + 1
BB = 512


def _dot(a, b):
    return lax.dot_general(a, b, (((1,), (0,)), ((), ())),
                           preferred_element_type=jnp.float32)


def _dense_body(dense_ref, sparse_ref, wb0_ref, bb0_ref, wb1_ref, bb1_ref,
                wb2_ref, bb2_ref, ahat_ref, wt0d_ref, bt0_ref, wt1_ref,
                bt1_ref, wt2_ref, bt2_ref, out_ref, xs_ref):
    xd = dense_ref[...]
    h = jnp.maximum(_dot(xd, wb0_ref[...]) + bb0_ref[...], 0.0)
    h = jnp.maximum(_dot(h, wb1_ref[...]) + bb1_ref[...], 0.0)
    dout = _dot(h, wb2_ref[...]) + bb2_ref[...]

    xs_ref[:, 0, :] = dout
    xs_ref[:, 1:, :] = sparse_ref[:, :N_FIELDS, :]
    x = xs_ref[...]

    g = lax.dot_general(x, x, (((2,), (2,)), ((0,), (0,))),
                        preferred_element_type=jnp.float32)

    t = _dot(dout, wt0d_ref[...]) + bt0_ref[...]
    for i in range(N_CONCAT):
        t += _dot(g[:, i, :], ahat_ref[i])
    h1 = jnp.maximum(t, 0.0)
    h2 = jnp.maximum(_dot(h1, wt1_ref[...]) + bt1_ref[...], 0.0)
    out_ref[...] = _dot(h2, wt2_ref[...]) + bt2_ref[...]


@jax.jit
def _dense_tail(dense_x, sparse_out, Wb0p, bb0, Wb1, bb1, Wb2, bb2,
                Ahat, Wt0d, bt0, Wt1, bt1, Wt2, bt2):
    nblk = dense_x.shape[0] // BB
    whole = lambda *shape: pl.BlockSpec(shape, lambda i: (0,) * len(shape))
    out = pl.pallas_call(
        _dense_body,
        grid=(nblk,),
        in_specs=[
            pl.BlockSpec((BB, 128), lambda i: (i, 0)),
            pl.BlockSpec((BB, N_SLOT, EMBED_DIM), lambda i: (i, 0, 0)),
            whole(128, 512), whole(1, 512),
            whole(512, 256), whole(1, 256),
            whole(256, 32), whole(1, 32),
            whole(N_CONCAT, N_CONCAT, 512),
            whole(32, 512), whole(1, 512),
            whole(512, 256), whole(1, 256),
            whole(256, 1), whole(1, 1),
        ],
        out_specs=pl.BlockSpec((BB, 1), lambda i: (i, 0)),
        out_shape=jax.ShapeDtypeStruct((dense_x.shape[0], 1), jnp.float32),
        scratch_shapes=[pltpu.VMEM((BB, N_CONCAT, EMBED_DIM), jnp.float32)],
    )(dense_x, sparse_out, Wb0p, bb0, Wb1, bb1, Wb2, bb2,
      Ahat, Wt0d, bt0, Wt1, bt1, Wt2, bt2)
    return out[:, 0]


N_SLOT = 32


def kernel(dense_x, sparse_x, embedding_table,
           Wb0, bb0, Wb1, bb1, Wb2, bb2,
           Wt0, bt0, Wt1, bt1, Wt2, bt2):
    offsets = jnp.asarray(np.concatenate([[0], np.cumsum(NUM_EMB)[:-1]]),
                          dtype=sparse_x.dtype)
    indices = (sparse_x + offsets[None, :]).astype(jnp.int32)
    idx32 = jnp.pad(indices, ((0, 0), (0, N_SLOT - N_FIELDS)))
    sizes = [1024] * 4
    bounds = np.concatenate([[0], np.cumsum(sizes)])
    sparse_h = [embedding_table.at[idx32[bounds[h]:bounds[h + 1]]].get(
        mode="promise_in_bounds") for h in range(len(sizes))]

    dense_xp = jnp.pad(dense_x, ((0, 0), (0, 128 - dense_x.shape[1])))
    Wb0p = jnp.pad(Wb0, ((0, 128 - Wb0.shape[0]), (0, 0)))

    iu = np.triu_indices(N_CONCAT, k=1)
    P = np.zeros((N_CONCAT * N_CONCAT, 351), np.float32)
    P[27 * iu[0] + iu[1], np.arange(351)] = 0.5
    P[27 * iu[1] + iu[0], np.arange(351)] = 0.5
    Ahat = (jnp.asarray(P) @ Wt0[:351]).reshape(N_CONCAT, N_CONCAT, 512)
    Wt0d = Wt0[351:]

    outs = [
        _dense_tail(dense_xp[bounds[h]:bounds[h + 1]], sparse_h[h], Wb0p,
                    bb0[None, :], Wb1, bb1[None, :], Wb2, bb2[None, :],
                    Ahat, Wt0d, bt0[None, :], Wt1, bt1[None, :], Wt2,
                    bt2[None, :])
        for h in range(len(sizes))
    ]
    return jnp.concatenate(outs)
